# baseline (speedup 1.0000x reference)
import functools

import jax
import jax.numpy as jnp
from jax.experimental import pallas as pl
from jax.experimental.pallas import tpu as pltpu

N = 4096
D = 128
H = 64
BLK = 512
NB = N // BLK


def _gnn_kernel(x_ref, adj_ref, wr0, br0, wo0, wr1, br1, wo1, wr2, br2, wo2,
                out_ref, adj_bf, g_s, g1_s, *accs):
    s = pl.program_id(0)

    def dot(a, b):
        return jax.lax.dot_general(a, b, (((0,), (0,)), ((), ())),
                                   preferred_element_type=jnp.float32)

    def rowdot(a, b):
        return jax.lax.dot_general(a, b, (((1,), (1,)), ((), ())),
                                   preferred_element_type=jnp.float32)

    @pl.when(s == 0)
    def _():
        g_s[...] = rowdot(x_ref[...], wr0[...]).astype(jnp.bfloat16)

    for st in range(NB):
        @pl.when(s == st)
        def _(st=st):
            lo, hi = st * BLK, (st + 1) * BLK
            a = adj_ref[...].astype(jnp.bfloat16)
            adj_bf[st] = a
            agg = dot(a, g_s[...])
            root = rowdot(x_ref[lo:hi, :], wo0[...])
            res = jnp.maximum(agg + root + br0[...], 0.0)
            g1_s[lo:hi, :] = rowdot(res, wr1[...]).astype(jnp.bfloat16)
            accs[st][...] = rowdot(res, wo1[...]) + br1[...]
            if st > 0:
                plo, phi = (st - 1) * BLK, st * BLK
                accs[st - 1][...] += dot(adj_bf[st - 1, :phi, :],
                                         g1_s[:phi, :])
                for i in range(st - 1):
                    accs[i][...] += dot(
                        adj_bf[i, plo:phi, :], g1_s[plo:phi, :])

    @pl.when(s == NB)
    def _():
        plo = (NB - 1) * BLK
        accs[NB - 1][...] += dot(adj_bf[NB - 1], g1_s[...])
        for i in range(NB - 1):
            accs[i][...] += dot(adj_bf[i, plo:, :], g1_s[plo:, :])
        for i in range(NB):
            lo, hi = i * BLK, (i + 1) * BLK
            h2 = jnp.maximum(accs[i][...], 0.0)
            accs[i][...] = h2
            g_s[lo:hi, :] = rowdot(h2, wr2[...]).astype(jnp.bfloat16)

    @pl.when(s == NB + 1)
    def _():
        for i in range(NB):
            lo, hi = i * BLK, (i + 1) * BLK
            agg = dot(adj_bf[i], g_s[...])
            root = rowdot(accs[i][...], wo2[...])
            out_ref[lo:hi, :] = jnp.maximum(agg + root + br2[...], 0.0)


@functools.partial(jax.jit, static_argnames=("interpret",))
def _run(X, adj_mat, W_rel0, b_rel0, W_root0, W_rel1, b_rel1, W_root1,
         W_rel2, b_rel2, W_root2, interpret=False):
    b0 = b_rel0.reshape(1, H)
    b1 = b_rel1.reshape(1, H)
    b2 = b_rel2.reshape(1, H)
    xb = X.astype(jnp.bfloat16)
    full = lambda shape: pl.BlockSpec(shape, lambda s: (0,) * len(shape))
    return pl.pallas_call(
        _gnn_kernel,
        grid=(NB + 2,),
        in_specs=[
            full((N, D)),
            pl.BlockSpec((N, BLK),
                         lambda s: (0, jnp.minimum(s, NB - 1))),
            full((H, D)), full((1, H)), full((H, D)),
            full((H, H)), full((1, H)), full((H, H)),
            full((H, H)), full((1, H)), full((H, H)),
        ],
        out_specs=full((N, H)),
        out_shape=jax.ShapeDtypeStruct((N, H), jnp.float32),
        scratch_shapes=[
            pltpu.VMEM((NB, N, BLK), jnp.bfloat16),
            pltpu.VMEM((N, H), jnp.bfloat16),
            pltpu.VMEM((N, H), jnp.bfloat16),
        ] + [pltpu.VMEM((BLK, H), jnp.float32) for _ in range(NB)],
        interpret=interpret,
    )(xb, adj_mat, W_rel0.astype(jnp.bfloat16), b0,
      W_root0.astype(jnp.bfloat16), W_rel1, b1, W_root1, W_rel2, b2, W_root2)


def kernel(X, adj_mat, W_rel0, b_rel0, W_root0, W_rel1, b_rel1, W_root1,
           W_rel2, b_rel2, W_root2):
    return _run(X, adj_mat, W_rel0, b_rel0, W_root0, W_rel1, b_rel1, W_root1,
                W_rel2, b_rel2, W_root2)


# speedup vs baseline: 1.0641x; 1.0641x over previous
import functools

import jax
import jax.numpy as jnp
from jax.experimental import pallas as pl
from jax.experimental.pallas import tpu as pltpu

N = 4096
D = 128
H = 64
BLK = 512
NB = N // BLK


def _gnn_kernel(x_ref, adj_ref, wr0, br0, wo0, wr1, br1, wo1, wr2, br2, wo2,
                out_ref, adj_bf, h_s, g_s):
    s = pl.program_id(0)

    @pl.when(s == 0)
    def _():
        g = jax.lax.dot_general(x_ref[...], wr0[...],
                                (((1,), (1,)), ((), ())),
                                preferred_element_type=jnp.float32)
        g_s[...] = g.astype(jnp.bfloat16)

    HB = BLK // 2

    @pl.when(s < NB)
    def _():
        for hh in range(2):
            a = adj_ref[:, hh * HB:(hh + 1) * HB].astype(jnp.bfloat16)
            adj_bf[s, :, hh * HB:(hh + 1) * HB] = a
            agg = jax.lax.dot_general(a, g_s[...],
                                      (((0,), (0,)), ((), ())),
                                      preferred_element_type=jnp.float32)
            x_blk = x_ref[pl.ds(s * BLK + hh * HB, HB), :]
            root = jax.lax.dot_general(x_blk, wo0[...],
                                       (((1,), (1,)), ((), ())),
                                       preferred_element_type=jnp.float32)
            res = jnp.maximum(agg + root + br0[...], 0.0)
            h_s[pl.ds(s * BLK + hh * HB, HB), :] = res

    def layer(wr, br, wo, last):
        g = jax.lax.dot_general(h_s[...], wr[...],
                                (((1,), (1,)), ((), ())),
                                preferred_element_type=jnp.float32)
        g_s[...] = g.astype(jnp.bfloat16)
        for i in range(NB):
            agg = jax.lax.dot_general(adj_bf[i], g_s[...],
                                      (((0,), (0,)), ((), ())),
                                      preferred_element_type=jnp.float32)
            h_blk = h_s[i * BLK:(i + 1) * BLK, :]
            root = jax.lax.dot_general(h_blk, wo[...],
                                       (((1,), (1,)), ((), ())),
                                       preferred_element_type=jnp.float32)
            res = jnp.maximum(agg + root + br[...], 0.0)
            if last:
                out_ref[i * BLK:(i + 1) * BLK, :] = res
            else:
                h_s[i * BLK:(i + 1) * BLK, :] = res

    @pl.when(s == NB)
    def _():
        layer(wr1, br1, wo1, last=False)

    @pl.when(s == NB + 1)
    def _():
        layer(wr2, br2, wo2, last=True)


@functools.partial(jax.jit, static_argnames=("interpret",))
def _run(X, adj_mat, W_rel0, b_rel0, W_root0, W_rel1, b_rel1, W_root1,
         W_rel2, b_rel2, W_root2, interpret=False):
    b0 = b_rel0.reshape(1, H)
    b1 = b_rel1.reshape(1, H)
    b2 = b_rel2.reshape(1, H)
    full = lambda shape: pl.BlockSpec(shape, lambda s: (0,) * len(shape))
    return pl.pallas_call(
        _gnn_kernel,
        grid=(NB + 2,),
        in_specs=[
            full((N, D)),
            pl.BlockSpec((N, BLK),
                         lambda s: (0, jnp.minimum(s, NB - 1))),
            full((H, D)), full((1, H)), full((H, D)),
            full((H, H)), full((1, H)), full((H, H)),
            full((H, H)), full((1, H)), full((H, H)),
        ],
        out_specs=full((N, H)),
        out_shape=jax.ShapeDtypeStruct((N, H), jnp.float32),
        scratch_shapes=[
            pltpu.VMEM((NB, N, BLK), jnp.bfloat16),
            pltpu.VMEM((N, H), jnp.float32),
            pltpu.VMEM((N, H), jnp.bfloat16),
        ],
        interpret=interpret,
    )(X, adj_mat, W_rel0, b0, W_root0, W_rel1, b1, W_root1, W_rel2, b2, W_root2)


def kernel(X, adj_mat, W_rel0, b_rel0, W_root0, W_rel1, b_rel1, W_root1,
           W_rel2, b_rel2, W_root2):
    return _run(X, adj_mat, W_rel0, b_rel0, W_root0, W_rel1, b_rel1, W_root1,
                W_rel2, b_rel2, W_root2)
